# baseline (speedup 1.0000x reference)
import functools

import jax
import jax.numpy as jnp
from jax import lax
from jax.experimental import pallas as pl
from jax.experimental.pallas import tpu as pltpu
from jax.experimental.pallas import tpu_sc as plsc

NUM_EMBEDDINGS = 100000
EMBEDDING_DIM = 128
BATCH = 4096
HIST = 50

_INFO = plsc.get_sparse_core_info()
_NC = _INFO.num_cores
_NS = _INFO.num_subcores
_NW = _NC * _NS
_B_PER_W = BATCH // _NW
_NB = 7
_G = 3


def _sc_gather(idx_hbm, table_hbm):
    mesh = plsc.VectorSubcoreMesh(core_axis_name="c", subcore_axis_name="s")

    @functools.partial(
        pl.kernel,
        mesh=mesh,
        out_type=jax.ShapeDtypeStruct((HIST, BATCH, EMBEDDING_DIM), jnp.float32),
        scratch_types=[
            pltpu.VMEM((HIST, _B_PER_W), jnp.int32),
            pltpu.VMEM((_NB, _B_PER_W, EMBEDDING_DIM), jnp.float32),
            pltpu.SemaphoreType.DMA((_NB,)),
            pltpu.SemaphoreType.DMA((_NB,)),
        ],
    )
    def k(idx_ref, table_ref, out_ref, idx_v, rows_v, gsem, ssem):
        wid = lax.axis_index("s") * _NC + lax.axis_index("c")
        base = wid * _B_PER_W
        pltpu.sync_copy(idx_ref.at[:, pl.ds(base, _B_PER_W)], idx_v)

        def step(j, carry):
            @pl.when(j < HIST)
            def _fire():
                b = j % _NB

                @pl.when(j >= _NB)
                def _reuse():
                    pltpu.make_async_copy(
                        rows_v.at[b], out_ref.at[0, pl.ds(base, _B_PER_W)], ssem.at[b]
                    ).wait()

                pltpu.async_copy(table_ref.at[idx_v.at[j]], rows_v.at[b], gsem.at[b])

            @pl.when(j >= _G)
            def _drain():
                c = j - _G
                b = c % _NB
                pltpu.make_async_copy(
                    table_ref.at[idx_v.at[c]], rows_v.at[b], gsem.at[b]
                ).wait()
                pltpu.async_copy(
                    rows_v.at[b], out_ref.at[c, pl.ds(base, _B_PER_W)], ssem.at[b]
                )

            return carry

        lax.fori_loop(0, HIST + _G, step, 0)

        for b in range(_NB):
            pltpu.make_async_copy(
                rows_v.at[b], out_ref.at[0, pl.ds(base, _B_PER_W)], ssem.at[b]
            ).wait()

    return k(idx_hbm, table_hbm)


def kernel(input_, weight):
    idx_t = input_.astype(jnp.int32).T
    out_hm = _sc_gather(idx_t, weight)
    return jnp.transpose(out_hm, (1, 0, 2))


# speedup vs baseline: 1.0052x; 1.0052x over previous
import functools

import jax
import jax.numpy as jnp
from jax import lax
from jax.experimental import pallas as pl
from jax.experimental.pallas import tpu as pltpu
from jax.experimental.pallas import tpu_sc as plsc

NUM_EMBEDDINGS = 100000
EMBEDDING_DIM = 128
BATCH = 4096
HIST = 50

_INFO = plsc.get_sparse_core_info()
_NC = _INFO.num_cores
_NS = _INFO.num_subcores
_NW = _NC * _NS
_B_PER_W = BATCH // _NW
_NB = 6
_G = 3


def _sc_gather(idx_hbm, table_hbm):
    mesh = plsc.VectorSubcoreMesh(core_axis_name="c", subcore_axis_name="s")

    @functools.partial(
        pl.kernel,
        mesh=mesh,
        out_type=jax.ShapeDtypeStruct((HIST, BATCH, EMBEDDING_DIM), jnp.float32),
        scratch_types=[
            pltpu.VMEM((HIST, _B_PER_W), jnp.int32),
            pltpu.VMEM((_NB, _B_PER_W, EMBEDDING_DIM), jnp.float32),
            pltpu.SemaphoreType.DMA((_NB,)),
            pltpu.SemaphoreType.DMA((_NB,)),
        ],
    )
    def k(idx_ref, table_ref, out_ref, idx_v, rows_v, gsem, ssem):
        wid = lax.axis_index("s") * _NC + lax.axis_index("c")
        base = wid * _B_PER_W
        pltpu.sync_copy(idx_ref.at[:, pl.ds(base, _B_PER_W)], idx_v)

        def step(j, carry):
            @pl.when(j < HIST)
            def _fire():
                b = j % _NB

                @pl.when(j >= _NB)
                def _reuse():
                    pltpu.make_async_copy(
                        rows_v.at[b], out_ref.at[0, pl.ds(base, _B_PER_W)], ssem.at[b]
                    ).wait()

                pltpu.async_copy(table_ref.at[idx_v.at[j]], rows_v.at[b], gsem.at[b])

            @pl.when(j >= _G)
            def _drain():
                c = j - _G
                b = c % _NB
                pltpu.make_async_copy(
                    table_ref.at[idx_v.at[c]], rows_v.at[b], gsem.at[b]
                ).wait()
                pltpu.async_copy(
                    rows_v.at[b], out_ref.at[c, pl.ds(base, _B_PER_W)], ssem.at[b]
                )

            return carry

        lax.fori_loop(0, HIST + _G, step, 0)

        for b in range(_NB):
            pltpu.make_async_copy(
                rows_v.at[b], out_ref.at[0, pl.ds(base, _B_PER_W)], ssem.at[b]
            ).wait()

    return k(idx_hbm, table_hbm)


def kernel(input_, weight):
    idx_t = input_.astype(jnp.int32).T
    out_hm = _sc_gather(idx_t, weight)
    return jnp.transpose(out_hm, (1, 0, 2))


# speedup vs baseline: 1.0058x; 1.0006x over previous
import functools

import jax
import jax.numpy as jnp
from jax import lax
from jax.experimental import pallas as pl
from jax.experimental.pallas import tpu as pltpu
from jax.experimental.pallas import tpu_sc as plsc

NUM_EMBEDDINGS = 100000
EMBEDDING_DIM = 128
BATCH = 4096
HIST = 50

_INFO = plsc.get_sparse_core_info()
_NC = _INFO.num_cores
_NS = _INFO.num_subcores
_NW = _NC * _NS
_B_PER_W = BATCH // _NW
_NB = 6
_G = 3


def _sc_gather(idx_hbm, table_hbm):
    mesh = plsc.VectorSubcoreMesh(core_axis_name="c", subcore_axis_name="s")

    @functools.partial(
        pl.kernel,
        mesh=mesh,
        out_type=jax.ShapeDtypeStruct((HIST, BATCH, EMBEDDING_DIM), jnp.float32),
        compiler_params=pltpu.CompilerParams(
            disable_bounds_checks=True, disable_semaphore_checks=True
        ),
        scratch_types=[
            pltpu.VMEM((HIST, _B_PER_W), jnp.int32),
            pltpu.VMEM((_NB, _B_PER_W, EMBEDDING_DIM), jnp.float32),
            pltpu.SemaphoreType.DMA((_NB,)),
            pltpu.SemaphoreType.DMA((_NB,)),
        ],
    )
    def k(idx_ref, table_ref, out_ref, idx_v, rows_v, gsem, ssem):
        wid = lax.axis_index("s") * _NC + lax.axis_index("c")
        base = wid * _B_PER_W
        pltpu.sync_copy(idx_ref.at[:, pl.ds(base, _B_PER_W)], idx_v)

        def step(j, carry):
            @pl.when(j < HIST)
            def _fire():
                b = j % _NB

                @pl.when(j >= _NB)
                def _reuse():
                    pltpu.make_async_copy(
                        rows_v.at[b], out_ref.at[0, pl.ds(base, _B_PER_W)], ssem.at[b]
                    ).wait()

                pltpu.async_copy(table_ref.at[idx_v.at[j]], rows_v.at[b], gsem.at[b])

            @pl.when(j >= _G)
            def _drain():
                c = j - _G
                b = c % _NB
                pltpu.make_async_copy(
                    table_ref.at[idx_v.at[c]], rows_v.at[b], gsem.at[b]
                ).wait()
                pltpu.async_copy(
                    rows_v.at[b], out_ref.at[c, pl.ds(base, _B_PER_W)], ssem.at[b]
                )

            return carry

        lax.fori_loop(0, HIST + _G, step, 0)

        for b in range(_NB):
            pltpu.make_async_copy(
                rows_v.at[b], out_ref.at[0, pl.ds(base, _B_PER_W)], ssem.at[b]
            ).wait()

    return k(idx_hbm, table_hbm)


def kernel(input_, weight):
    idx_t = input_.astype(jnp.int32).T
    out_hm = _sc_gather(idx_t, weight)
    return jnp.transpose(out_hm, (1, 0, 2))
